# initial kernel scaffold (unmeasured)
import jax
import jax.numpy as jnp
from jax import lax
from jax.experimental import pallas as pl
from jax.experimental.pallas import tpu as pltpu

N_DEV = 4
E_LOCAL = 4
E_TOT = 16


def kernel(x, router_W, route_idx, expert_W, shared_W):
    n_tok, d_model = x.shape
    d_hidden = expert_W.shape[-1]

    def body(x_ref, rw_ref, idx_ref, ew_ref, sw_ref, out_ref,
             x_stage, recv_x, recv_idx, y_stage, recv_y,
             sem_sx, sem_rx, sem_si, sem_ri, sem_sy, sem_ry):
        me = lax.axis_index("i")

        bar = pltpu.get_barrier_semaphore()
        for d in range(1, N_DEV):
            pl.semaphore_signal(
                bar, inc=1,
                device_id=((me + d) % N_DEV,),
                device_id_type=pl.DeviceIdType.MESH,
            )
        pl.semaphore_wait(bar, N_DEV - 1)

        x_bf = x_ref[...].astype(jnp.bfloat16)
        x_stage[...] = x_bf
        sends = []
        for d in range(1, N_DEV):
            q = (me + d) % N_DEV
            r = pltpu.make_async_remote_copy(
                src_ref=x_stage, dst_ref=recv_x.at[me],
                send_sem=sem_sx.at[q], recv_sem=sem_rx.at[me],
                device_id=(q,), device_id_type=pl.DeviceIdType.MESH,
            )
            r.start()
            sends.append(r)
            r = pltpu.make_async_remote_copy(
                src_ref=idx_ref, dst_ref=recv_idx.at[me],
                send_sem=sem_si.at[q], recv_sem=sem_ri.at[me],
                device_id=(q,), device_id_type=pl.DeviceIdType.MESH,
            )
            r.start()
            sends.append(r)

        rw_bf = rw_ref[...].astype(jnp.bfloat16)
        w_stack = ew_ref[...].astype(jnp.bfloat16).reshape(E_LOCAL * d_model,
                                                           d_hidden)

        def contrib(xb, idx):
            s = jnp.dot(xb, rw_bf, preferred_element_type=jnp.float32)
            s = s - jnp.max(s, axis=1, keepdims=True)
            p = jnp.exp(s)
            p = p / jnp.sum(p, axis=1, keepdims=True)
            eids = lax.broadcasted_iota(jnp.int32, (n_tok, E_TOT), 1)
            gate = jnp.sum(jnp.where(eids == idx, p, 0.0), axis=1,
                           keepdims=True)
            cols = []
            for k in range(E_LOCAL):
                ck = jnp.where(idx == me * E_LOCAL + k, gate, 0.0)
                cols.append(xb * ck.astype(jnp.bfloat16))
            xm = jnp.concatenate(cols, axis=1)
            return jnp.dot(xm, w_stack, preferred_element_type=jnp.float32)

        sw_bf = sw_ref[...].astype(jnp.bfloat16)
        shared = jnp.dot(x_bf, sw_bf, preferred_element_type=jnp.float32)
        out_ref[...] = shared + contrib(x_bf, idx_ref[...])

        for d in (1, 3, 2):
            q = (me + d) % N_DEV
            for buf, sem in ((recv_x, sem_rx), (recv_idx, sem_ri)):
                w = pltpu.make_async_remote_copy(
                    src_ref=buf.at[q], dst_ref=buf.at[q],
                    send_sem=sem.at[q], recv_sem=sem.at[q],
                    device_id=(q,), device_id_type=pl.DeviceIdType.MESH,
                )
                w.wait_recv()
            y_stage[q, :, :] = contrib(recv_x[q], recv_idx[q]).astype(
                jnp.bfloat16)
            r = pltpu.make_async_remote_copy(
                src_ref=y_stage.at[q], dst_ref=recv_y.at[me],
                send_sem=sem_sy.at[q], recv_sem=sem_ry.at[me],
                device_id=(q,), device_id_type=pl.DeviceIdType.MESH,
            )
            r.start()
            sends.append(r)

        acc = out_ref[...]
        for d in (1, 2, 3):
            q = (me + d) % N_DEV
            w = pltpu.make_async_remote_copy(
                src_ref=recv_y.at[q], dst_ref=recv_y.at[q],
                send_sem=sem_ry.at[q], recv_sem=sem_ry.at[q],
                device_id=(q,), device_id_type=pl.DeviceIdType.MESH,
            )
            w.wait_recv()
            acc = acc + recv_y[q].astype(jnp.float32)
        out_ref[...] = acc

        for r in sends:
            r.wait_send()

    return pl.pallas_call(
        body,
        out_shape=jax.ShapeDtypeStruct((n_tok, d_hidden), jnp.float32),
        in_specs=[pl.BlockSpec(memory_space=pltpu.VMEM)] * 5,
        out_specs=pl.BlockSpec(memory_space=pltpu.VMEM),
        scratch_shapes=[
            pltpu.VMEM((n_tok, d_model), jnp.bfloat16),
            pltpu.VMEM((N_DEV, n_tok, d_model), jnp.bfloat16),
            pltpu.VMEM((N_DEV, n_tok, 1), jnp.int32),
            pltpu.VMEM((N_DEV, n_tok, d_hidden), jnp.bfloat16),
            pltpu.VMEM((N_DEV, n_tok, d_hidden), jnp.bfloat16),
            pltpu.SemaphoreType.DMA((N_DEV,)),
            pltpu.SemaphoreType.DMA((N_DEV,)),
            pltpu.SemaphoreType.DMA((N_DEV,)),
            pltpu.SemaphoreType.DMA((N_DEV,)),
            pltpu.SemaphoreType.DMA((N_DEV,)),
            pltpu.SemaphoreType.DMA((N_DEV,)),
        ],
        compiler_params=pltpu.CompilerParams(collective_id=0),
    )(x, router_W, route_idx, expert_W, shared_W)


# baseline (device time: 115287 ns/iter reference)
import jax
import jax.numpy as jnp
from jax import lax
from jax.experimental import pallas as pl
from jax.experimental.pallas import tpu as pltpu

N_DEV = 4
E_LOCAL = 4
E_TOT = 16


def kernel(x, router_W, route_idx, expert_W, shared_W):
    n_tok, d_model = x.shape
    d_hidden = expert_W.shape[-1]

    def body(x_ref, rw_ref, idx_ref, ew_ref, sw_ref, out_ref,
             x_stage, recv_x, recv_idx, y_stage, recv_y,
             sem_sx, sem_rx, sem_si, sem_ri, sem_sy, sem_ry):
        me = lax.axis_index("i")

        bar = pltpu.get_barrier_semaphore()
        for d in range(1, N_DEV):
            pl.semaphore_signal(
                bar, inc=1,
                device_id=((me + d) % N_DEV,),
                device_id_type=pl.DeviceIdType.MESH,
            )
        pl.semaphore_wait(bar, N_DEV - 1)

        x_bf = x_ref[...].astype(jnp.bfloat16)
        x_stage[...] = x_bf
        sends = []
        for d in range(1, N_DEV):
            q = (me + d) % N_DEV
            r = pltpu.make_async_remote_copy(
                src_ref=x_stage, dst_ref=recv_x.at[me],
                send_sem=sem_sx.at[q], recv_sem=sem_rx.at[me],
                device_id=(q,), device_id_type=pl.DeviceIdType.MESH,
            )
            r.start()
            sends.append(r)
            r = pltpu.make_async_remote_copy(
                src_ref=idx_ref, dst_ref=recv_idx.at[me],
                send_sem=sem_si.at[q], recv_sem=sem_ri.at[me],
                device_id=(q,), device_id_type=pl.DeviceIdType.MESH,
            )
            r.start()
            sends.append(r)

        rw_bf = rw_ref[...].astype(jnp.bfloat16)
        w_stack = ew_ref[...].astype(jnp.bfloat16).reshape(E_LOCAL * d_model,
                                                           d_hidden)

        def contrib(xb, idx):
            s = jnp.dot(xb, rw_bf, preferred_element_type=jnp.float32)
            s = s - jnp.max(s, axis=1, keepdims=True)
            p = jnp.exp(s)
            p = p / jnp.sum(p, axis=1, keepdims=True)
            eids = lax.broadcasted_iota(jnp.int32, (n_tok, E_TOT), 1)
            gate = jnp.sum(jnp.where(eids == idx, p, 0.0), axis=1,
                           keepdims=True)
            cols = []
            for k in range(E_LOCAL):
                ck = jnp.where(idx == me * E_LOCAL + k, gate, 0.0)
                cols.append(xb * ck.astype(jnp.bfloat16))
            xm = jnp.concatenate(cols, axis=1)
            return jnp.dot(xm, w_stack, preferred_element_type=jnp.float32)

        sw_bf = sw_ref[...].astype(jnp.bfloat16)
        shared = jnp.dot(x_bf, sw_bf, preferred_element_type=jnp.float32)
        out_ref[...] = shared + contrib(x_bf, idx_ref[...])

        for d in (1, 3, 2):
            q = (me + d) % N_DEV
            for buf, sem in ((recv_x, sem_rx), (recv_idx, sem_ri)):
                w = pltpu.make_async_remote_copy(
                    src_ref=buf.at[q], dst_ref=buf.at[q],
                    send_sem=sem.at[q], recv_sem=sem.at[q],
                    device_id=(q,), device_id_type=pl.DeviceIdType.MESH,
                )
                w.wait_recv()
            y_stage[q, :, :] = contrib(recv_x[q], recv_idx[q]).astype(
                jnp.bfloat16)
            r = pltpu.make_async_remote_copy(
                src_ref=y_stage.at[q], dst_ref=recv_y.at[me],
                send_sem=sem_sy.at[q], recv_sem=sem_ry.at[me],
                device_id=(q,), device_id_type=pl.DeviceIdType.MESH,
            )
            r.start()
            sends.append(r)

        acc = out_ref[...]
        for d in (1, 2, 3):
            q = (me + d) % N_DEV
            w = pltpu.make_async_remote_copy(
                src_ref=recv_y.at[q], dst_ref=recv_y.at[q],
                send_sem=sem_ry.at[q], recv_sem=sem_ry.at[q],
                device_id=(q,), device_id_type=pl.DeviceIdType.MESH,
            )
            w.wait_recv()
            acc = acc + recv_y[q].astype(jnp.float32)
        out_ref[...] = acc

        for r in sends:
            r.wait_send()

    return pl.pallas_call(
        body,
        out_shape=jax.ShapeDtypeStruct((n_tok, d_hidden), jnp.float32),
        in_specs=[pl.BlockSpec(memory_space=pltpu.VMEM)] * 5,
        out_specs=pl.BlockSpec(memory_space=pltpu.VMEM),
        scratch_shapes=[
            pltpu.VMEM((n_tok, d_model), jnp.bfloat16),
            pltpu.VMEM((N_DEV, n_tok, d_model), jnp.bfloat16),
            pltpu.VMEM((N_DEV, n_tok, 1), jnp.int32),
            pltpu.VMEM((N_DEV, n_tok, d_hidden), jnp.bfloat16),
            pltpu.VMEM((N_DEV, n_tok, d_hidden), jnp.bfloat16),
            pltpu.SemaphoreType.DMA((N_DEV,)),
            pltpu.SemaphoreType.DMA((N_DEV,)),
            pltpu.SemaphoreType.DMA((N_DEV,)),
            pltpu.SemaphoreType.DMA((N_DEV,)),
            pltpu.SemaphoreType.DMA((N_DEV,)),
            pltpu.SemaphoreType.DMA((N_DEV,)),
        ],
        compiler_params=pltpu.CompilerParams(
            collective_id=0, vmem_limit_bytes=100 * 1024 * 1024,
        ),
    )(x, router_W, route_idx, expert_W, shared_W)
